# baseline (speedup 1.0000x reference)
import functools
import jax
import jax.numpy as jnp
from jax import lax
from jax.experimental import pallas as pl
from jax.experimental.pallas import tpu as pltpu
from jax.experimental.pallas import tpu_sc as plsc

LANE = 16
NC, NS = 2, 16
NW = NC * NS

HALF = 100


def _rsqrt_newton(xv):
    i = lax.bitcast_convert_type(xv, jnp.int32)
    i = jnp.int32(0x5F3759DF) - (i >> 1)
    y = lax.bitcast_convert_type(i, jnp.float32)
    nxh = xv * jnp.float32(-0.5)
    for _ in range(3):
        y = y * (jnp.float32(1.5) + nxh * y * y)
    return y


_GDN = lax.GatherDimensionNumbers(
    offset_dims=(), collapsed_slice_dims=(0,), start_index_map=(0,))


def _permute(v, p):
    return lax.gather(v, p[:, None], _GDN, slice_sizes=(1,),
                      mode=lax.GatherScatterMode.PROMISE_IN_BOUNDS)


def _lane_allreduce(v, perms):
    for p in perms:
        v = v + _permute(v, p)
    return v


def _make_kernel(B, L, V, E, MAXSEQ):
    n_rows = B * L
    assert L == 2 * HALF and E == 4 * LANE
    n_seq = B
    assert n_seq % NW == 0
    seq_per_w = n_seq // NW
    inv_e = jnp.float32(1.0 / E)
    KV = E // LANE

    mesh = plsc.VectorSubcoreMesh(core_axis_name="c", subcore_axis_name="s")

    @functools.partial(
        pl.kernel,
        out_type=jax.ShapeDtypeStruct((B, L, E), jnp.float32),
        mesh=mesh,
        compiler_params=pltpu.CompilerParams(use_tc_tiling_on_sc=False),
        scratch_types=[
            pltpu.VMEM((L, E), jnp.float32),
            pltpu.VMEM((E,), jnp.float32),
            pltpu.VMEM((E,), jnp.float32),
            pltpu.VMEM((2, HALF), jnp.int32),
            pltpu.VMEM((2, HALF), jnp.int32),
            pltpu.VMEM((L, E), jnp.float32),
            pltpu.VMEM((L, E), jnp.float32),
            pltpu.VMEM((L, E), jnp.float32),
            pltpu.VMEM((L, E), jnp.float32),
            pltpu.SemaphoreType.DMA,
            pltpu.SemaphoreType.DMA,
            pltpu.SemaphoreType.DMA,
            pltpu.SemaphoreType.DMA,
        ],
    )
    def emb(ids_hbm, tok_hbm, pos_hbm, g_hbm, be_hbm, out_hbm,
            pos_v, g_v, b_v, idx0, idx1, rows0, rows1, ob0, ob1,
            gs0, gs1, ss0, ss1):
        wid = lax.axis_index("s") * NC + lax.axis_index("c")
        seq0 = wid * seq_per_w

        idx = (idx0, idx1)
        rows = (rows0, rows1)
        ob = (ob0, ob1)
        gsem = (gs0, gs1)
        ssem = (ss0, ss1)

        pltpu.sync_copy(pos_hbm.at[pl.ds(0, L)], pos_v)
        pltpu.sync_copy(g_hbm, g_v)
        pltpu.sync_copy(be_hbm, b_v)

        g_r = [g_v[pl.ds(k * LANE, LANE)] for k in range(KV)]
        b_r = [b_v[pl.ds(k * LANE, LANE)] for k in range(KV)]

        def start_gather(s, b):
            pltpu.sync_copy(ids_hbm.at[seq0 + s], idx[b])
            pltpu.async_copy(tok_hbm.at[idx[b].at[0]],
                             rows[b].at[pl.ds(0, HALF)], gsem[b])
            pltpu.async_copy(tok_hbm.at[idx[b].at[1]],
                             rows[b].at[pl.ds(HALF, HALF)], gsem[b])

        def wait_gather(b):
            pltpu.make_async_copy(out_hbm.at[0], rows[b],
                                  gsem[b]).wait()

        def wait_store(b):
            pltpu.make_async_copy(ob[b], out_hbm.at[0],
                                  ssem[b]).wait()

        def start_store(s, b):
            pltpu.async_copy(ob[b], out_hbm.at[seq0 + s], ssem[b])

        U = 8

        iota = lax.iota(jnp.int32, LANE)
        perms = [jnp.bitwise_xor(iota, jnp.int32(d)) for d in (1, 2, 4, 8)]

        def compute(b):
            rv = rows[b]
            ov = ob[b]

            def row_block(j, carry):
                for u in range(U):
                    r = j * U + u
                    t = [rv[r, pl.ds(k * LANE, LANE)] +
                         pos_v[r, pl.ds(k * LANE, LANE)] for k in range(KV)]
                    ssum = (t[0] + t[1]) + (t[2] + t[3])
                    mean = _lane_allreduce(ssum, perms) * inv_e
                    q = (t[0] * t[0] + t[1] * t[1]) + \
                        (t[2] * t[2] + t[3] * t[3])
                    m2 = _lane_allreduce(q, perms) * inv_e
                    var = m2 - mean * mean
                    rs = _rsqrt_newton(var + jnp.float32(1e-12))
                    for k in range(KV):
                        a = rs * g_r[k]
                        off = b_r[k] - mean * a
                        ov[r, pl.ds(k * LANE, LANE)] = t[k] * a + off
                return carry

            lax.fori_loop(0, L // U, row_block, 0)

        start_gather(0, 0)
        start_gather(1, 1)

        def phase(s, b):
            wait_gather(b)

            @pl.when(s >= 2)
            def _():
                wait_store(b)

            compute(b)
            start_store(s, b)

            @pl.when(s + 2 < seq_per_w)
            def _():
                start_gather(s + 2, b)

        def outer(i2, carry):
            phase(i2 * 2, 0)
            phase(i2 * 2 + 1, 1)
            return carry

        lax.fori_loop(0, seq_per_w // 2, outer, 0)
        wait_store(0)
        wait_store(1)

    return emb


@jax.jit
def kernel(input_ids, token_table, pos_table, ln_gamma, ln_beta):
    B, L = input_ids.shape
    V, E = token_table.shape
    emb = _make_kernel(B, L, V, E, pos_table.shape[0])
    ids3 = input_ids.astype(jnp.int32).reshape(B, 2, HALF)
    return emb(ids3, token_table, pos_table, ln_gamma, ln_beta)


# speedup vs baseline: 1.3125x; 1.3125x over previous
import functools
import jax
import jax.numpy as jnp
from jax import lax
from jax.experimental import pallas as pl
from jax.experimental.pallas import tpu as pltpu
from jax.experimental.pallas import tpu_sc as plsc

LANE = 16
NC, NS = 2, 16
NW = NC * NS

HALF = 100
EPAD = 128


def _make_gather(B, L, V, E):
    n_rows = B * L
    assert L == 2 * HALF
    assert B % NW == 0
    seq_per_w = B // NW

    mesh = plsc.VectorSubcoreMesh(core_axis_name="c", subcore_axis_name="s")

    @functools.partial(
        pl.kernel,
        out_type=jax.ShapeDtypeStruct((n_rows, EPAD), jnp.float32),
        mesh=mesh,
        compiler_params=pltpu.CompilerParams(use_tc_tiling_on_sc=False),
        scratch_types=[
            pltpu.VMEM((2, HALF), jnp.int32),
            pltpu.VMEM((2, HALF), jnp.int32),
            pltpu.VMEM((L, E), jnp.float32),
            pltpu.VMEM((L, E), jnp.float32),
            pltpu.SemaphoreType.DMA,
            pltpu.SemaphoreType.DMA,
            pltpu.SemaphoreType.DMA,
            pltpu.SemaphoreType.DMA,
        ],
    )
    def gather(ids_hbm, tok_hbm, out_hbm,
               idx0, idx1, rows0, rows1, gs0, gs1, ss0, ss1):
        wid = lax.axis_index("s") * NC + lax.axis_index("c")
        seq0 = wid * seq_per_w

        idx = (idx0, idx1)
        rows = (rows0, rows1)
        gsem = (gs0, gs1)
        ssem = (ss0, ss1)

        def start_gather(s, b):
            pltpu.sync_copy(ids_hbm.at[seq0 + s], idx[b])
            pltpu.async_copy(tok_hbm.at[idx[b].at[0]],
                             rows[b].at[pl.ds(0, HALF)], gsem[b])
            pltpu.async_copy(tok_hbm.at[idx[b].at[1]],
                             rows[b].at[pl.ds(HALF, HALF)], gsem[b])

        def wait_gather(b):
            pltpu.make_async_copy(tok_hbm.at[pl.ds(0, L)], rows[b],
                                  gsem[b]).wait()

        def wait_store(b):
            pltpu.make_async_copy(rows[b],
                                  out_hbm.at[pl.ds(0, L), pl.ds(0, E)],
                                  ssem[b]).wait()

        def start_store(s, b):
            pltpu.async_copy(rows[b],
                             out_hbm.at[pl.ds((seq0 + s) * L, L),
                                        pl.ds(0, E)],
                             ssem[b])

        start_gather(0, 0)
        start_gather(1, 1)

        def phase(s, b):
            wait_gather(b)

            @pl.when(s >= 2)
            def _():
                wait_store(b)

            start_store(s, b)

            @pl.when(s + 2 < seq_per_w)
            def _():
                start_gather(s + 2, b)

        def outer(i2, carry):
            phase(i2 * 2, 0)
            phase(i2 * 2 + 1, 1)
            return carry

        lax.fori_loop(0, seq_per_w // 2, outer, 0)
        wait_store(0)
        wait_store(1)

    return gather


def _ln_block(x_ref, pos_ref, g_ref, b_ref, o_ref, *, bb, L, E):
    x = x_ref[...][:, :E].reshape(bb, L, E)
    x = x + pos_ref[...][None, :, :]
    mean = jnp.mean(x, axis=-1, keepdims=True)
    xc = x - mean
    var = jnp.mean(xc * xc, axis=-1, keepdims=True)
    y = xc * lax.rsqrt(var + jnp.float32(1e-12))
    o_ref[...] = y * g_ref[...][None, None, :] + b_ref[...][None, None, :]


def _make_ln(B, L, E, bb):
    grid = B // bb
    return pl.pallas_call(
        functools.partial(_ln_block, bb=bb, L=L, E=E),
        grid=(grid,),
        in_specs=[
            pl.BlockSpec((bb * L, EPAD), lambda i: (i, 0)),
            pl.BlockSpec((L, E), lambda i: (0, 0)),
            pl.BlockSpec((E,), lambda i: (0,)),
            pl.BlockSpec((E,), lambda i: (0,)),
        ],
        out_specs=pl.BlockSpec((bb, L, E), lambda i: (i, 0, 0)),
        out_shape=jax.ShapeDtypeStruct((B, L, E), jnp.float32),
    )


@jax.jit
def kernel(input_ids, token_table, pos_table, ln_gamma, ln_beta):
    B, L = input_ids.shape
    V, E = token_table.shape
    gather = _make_gather(B, L, V, E)
    ids3 = input_ids.astype(jnp.int32).reshape(B, 2, HALF)
    mid = gather(ids3, token_table)
    ln = _make_ln(B, L, E, bb=32)
    return ln(mid, pos_table[:L], ln_gamma, ln_beta)
